# baseline (device time: 33004 ns/iter reference)
import jax
import jax.numpy as jnp
from jax import lax
from jax.experimental import pallas as pl
from jax.experimental.pallas import tpu as pltpu

N_DEV = 8
B, D, H = 512, 256, 512
R = B // N_DEV
N_LAYERS = 3
MESH = pl.DeviceIdType.MESH
WIRE = jnp.bfloat16

GROUPS = ((0, 1), (1, 3), (3, 5), (5, 7), (7, 8))


def kernel(x, Win0, Wout0, Win1, Wout1, Win2, Wout2):
    def body(x_ref, win0, wout0, win1, wout1, win2, wout2, out_ref,
             partial_ref, rs_buf, x_buf, send_sems, recv_sems):
        my = lax.axis_index("i")

        def dev(idx):
            return (lax.rem(idx + N_DEV, N_DEV),)

        barrier_sem = pltpu.get_barrier_semaphore()
        for k in range(1, N_DEV):
            pl.semaphore_signal(barrier_sem, inc=1, device_id=dev(my + k),
                                device_id_type=MESH)
        pl.semaphore_wait(barrier_sem, N_DEV - 1)

        wins = [win0, win1, win2]
        wouts = [wout0, wout1, wout2]

        def mlp(xv, l):
            hv = jnp.dot(xv, wins[l][:, :], preferred_element_type=jnp.float32)
            hv = jnp.maximum(hv, 0.0)
            return jnp.dot(hv, wouts[l][:, :],
                           preferred_element_type=jnp.float32)

        def rs_recv_wait(l, slot):
            pltpu.make_async_remote_copy(
                src_ref=partial_ref.at[l, pl.ds(0, 1)],
                dst_ref=rs_buf.at[l, pl.ds(slot, 1)],
                send_sem=send_sems.at[l, 0, slot],
                recv_sem=recv_sems.at[l, 0, slot],
                device_id=dev(my), device_id_type=MESH,
            ).wait_recv()

        def finish_layer(l, own_f32):
            for j in range(1, N_DEV):
                rs_recv_wait(l, j - 1)
            acc = own_f32
            for s in range(N_DEV - 1):
                acc = acc + rs_buf[l, s].astype(jnp.float32)
            return acc

        p0 = mlp(x_ref[:, :], 0)
        partial_ref[0] = p0.astype(WIRE).reshape(N_DEV, R, D)
        for k in range(1, N_DEV):
            tgt = lax.rem(my + k, N_DEV)
            pltpu.make_async_remote_copy(
                src_ref=partial_ref.at[0, pl.ds(tgt, 1)],
                dst_ref=rs_buf.at[0, pl.ds(k - 1, 1)],
                send_sem=send_sems.at[0, 0, k - 1],
                recv_sem=recv_sems.at[0, 0, k - 1],
                device_id=dev(my + k), device_id_type=MESH,
            ).start()
        own0 = partial_ref[0, pl.ds(my, 1)][0].astype(jnp.float32)
        acc = finish_layer(0, own0)

        for l in range(1, N_LAYERS):
            x_buf[l - 1, 0] = acc.astype(WIRE)
            for k in range(1, N_DEV):
                pltpu.make_async_remote_copy(
                    src_ref=x_buf.at[l - 1, pl.ds(0, 1)],
                    dst_ref=x_buf.at[l - 1, pl.ds(k, 1)],
                    send_sem=send_sems.at[l - 1, 1, k - 1],
                    recv_sem=recv_sems.at[l - 1, 1, k - 1],
                    device_id=dev(my + k), device_id_type=MESH,
                ).start()

            own_f32 = None
            for (s, e) in GROUPS:
                for j in range(max(s, 1), e):
                    pltpu.make_async_remote_copy(
                        src_ref=x_buf.at[l - 1, pl.ds(0, 1)],
                        dst_ref=x_buf.at[l - 1, pl.ds(j, 1)],
                        send_sem=send_sems.at[l - 1, 1, j - 1],
                        recv_sem=recv_sems.at[l - 1, 1, j - 1],
                        device_id=dev(my - j), device_id_type=MESH,
                    ).wait_recv()
                xg = x_buf[l - 1, s:e].reshape((e - s) * R, D)
                pg = mlp(xg.astype(jnp.float32), l)
                partial_ref[l, s:e] = pg.astype(WIRE).reshape(e - s, R, D)
                if s == 0:
                    own_f32 = pg[0:R]
                for j in range(max(s, 1), e):
                    pltpu.make_async_remote_copy(
                        src_ref=partial_ref.at[l, pl.ds(j, 1)],
                        dst_ref=rs_buf.at[l, pl.ds(j - 1, 1)],
                        send_sem=send_sems.at[l, 0, j - 1],
                        recv_sem=recv_sems.at[l, 0, j - 1],
                        device_id=dev(my - j), device_id_type=MESH,
                    ).start()

            acc = finish_layer(l, own_f32)

        out_ref[:, :] = acc

        for l in range(N_LAYERS):
            for k in range(1, N_DEV):
                pltpu.make_async_remote_copy(
                    src_ref=partial_ref.at[l, pl.ds(0, 1)],
                    dst_ref=rs_buf.at[l, pl.ds(0, 1)],
                    send_sem=send_sems.at[l, 0, k - 1],
                    recv_sem=recv_sems.at[l, 0, k - 1],
                    device_id=dev(my), device_id_type=MESH,
                ).wait_send()
        for l in range(N_LAYERS - 1):
            for k in range(1, N_DEV):
                pltpu.make_async_remote_copy(
                    src_ref=x_buf.at[l, pl.ds(0, 1)],
                    dst_ref=x_buf.at[l, pl.ds(0, 1)],
                    send_sem=send_sems.at[l, 1, k - 1],
                    recv_sem=recv_sems.at[l, 1, k - 1],
                    device_id=dev(my), device_id_type=MESH,
                ).wait_send()

    return pl.pallas_call(
        body,
        out_shape=jax.ShapeDtypeStruct((R, D), jnp.float32),
        in_specs=[pl.BlockSpec(memory_space=pltpu.VMEM)] * 7,
        out_specs=pl.BlockSpec(memory_space=pltpu.VMEM),
        scratch_shapes=[
            pltpu.VMEM((N_LAYERS, N_DEV, R, D), WIRE),
            pltpu.VMEM((N_LAYERS, N_DEV - 1, R, D), WIRE),
            pltpu.VMEM((N_LAYERS - 1, N_DEV, R, D), WIRE),
            pltpu.SemaphoreType.DMA((N_LAYERS, 2, N_DEV - 1)),
            pltpu.SemaphoreType.DMA((N_LAYERS, 2, N_DEV - 1)),
        ],
        compiler_params=pltpu.CompilerParams(collective_id=0),
    )(x, Win0, Wout0, Win1, Wout1, Win2, Wout2)
